# Initial kernel scaffold; baseline (speedup 1.0000x reference)
import functools

import jax
import jax.numpy as jnp
from jax import lax
from jax.experimental import pallas as pl
from jax.experimental.pallas import tpu as pltpu
from jax.experimental.pallas import tpu_sc as plsc

N = 10000
E = 320000
D = 128
B = 64

NC = 2
NS = 16
NW = NC * NS
E_PER = E // NW
CH = 40
NCHUNK = E_PER // CH
ROWS_PER_TILE = N // NS
ZCH = 125

ROW_BLK = 2000
NB = N // ROW_BLK


def _make_sc_agg(with_deg: bool):
    mesh = plsc.VectorSubcoreMesh(core_axis_name="c", subcore_axis_name="s")

    out_type = [jax.ShapeDtypeStruct((NC, N, D), jnp.float32)]
    scratch = [
        pltpu.VMEM((NCHUNK, CH), jnp.int32),
        pltpu.VMEM((NCHUNK, CH), jnp.int32),
        pltpu.VMEM((CH, D), jnp.float32),
        pltpu.VMEM((CH, D), jnp.float32),
        pltpu.VMEM((ZCH, D), jnp.float32),
        pltpu.VMEM_SHARED((N, D), jnp.float32),
        pltpu.SemaphoreType.DMA,
        pltpu.SemaphoreType.DMA,
    ]
    if with_deg:
        out_type.append(jax.ShapeDtypeStruct((NC, N, 16), jnp.float32))
        scratch += [
            pltpu.VMEM((CH, 16), jnp.float32),
            pltpu.VMEM((ZCH, 16), jnp.float32),
            pltpu.VMEM_SHARED((N, 16), jnp.float32),
        ]

    def body(*refs):
        if with_deg:
            (x_hbm, srcr, dstr, z128, z16h, onesh, agg_o, deg_o,
             src_v, dst_v, rows0, rows1, zv, acc, s0, s1,
             ones_v, z16v, dacc) = refs
        else:
            (x_hbm, srcr, dstr, z128, agg_o,
             src_v, dst_v, rows0, rows1, zv, acc, s0, s1) = refs

        c = lax.axis_index("c")
        s = lax.axis_index("s")
        wid = s * NC + c
        base_row = s * ROWS_PER_TILE

        pltpu.sync_copy(z128, zv)
        for j in range(ROWS_PER_TILE // ZCH):
            pltpu.sync_copy(zv, acc.at[pl.ds(base_row + j * ZCH, ZCH)])
        if with_deg:
            pltpu.sync_copy(z16h, z16v)
            pltpu.sync_copy(onesh, ones_v)
            for j in range(ROWS_PER_TILE // ZCH):
                pltpu.sync_copy(z16v, dacc.at[pl.ds(base_row + j * ZCH, ZCH)])

        pltpu.sync_copy(srcr.at[wid], src_v)
        pltpu.sync_copy(dstr.at[wid], dst_v)

        plsc.subcore_barrier()

        def fire(chunk, buf, sem):
            pltpu.make_async_copy(x_hbm.at[src_v.at[chunk]], buf, sem).start()

        def drain(buf, sem):
            pltpu.make_async_copy(x_hbm.at[src_v.at[0]], buf, sem).wait()

        def put(chunk, buf):
            pltpu.sync_copy(buf, acc.at[dst_v.at[chunk]], add=True)
            if with_deg:
                pltpu.sync_copy(ones_v, dacc.at[dst_v.at[chunk]], add=True)

        fire(0, rows0, s0)

        def lbody(i, carry):
            a = 2 * i
            b = a + 1
            fire(b, rows1, s1)
            drain(rows0, s0)
            put(a, rows0)

            @pl.when(b + 1 < NCHUNK)
            def _():
                fire(b + 1, rows0, s0)

            drain(rows1, s1)
            put(b, rows1)
            return carry

        lax.fori_loop(0, NCHUNK // 2, lbody, 0)

        plsc.subcore_barrier()

        pltpu.sync_copy(acc.at[pl.ds(base_row, ROWS_PER_TILE)],
                        agg_o.at[c, pl.ds(base_row, ROWS_PER_TILE)])
        if with_deg:
            pltpu.sync_copy(dacc.at[pl.ds(base_row, ROWS_PER_TILE)],
                            deg_o.at[c, pl.ds(base_row, ROWS_PER_TILE)])

    return pl.kernel(body, out_type=out_type, mesh=mesh,
                     scratch_types=scratch)


_sc_agg_deg = _make_sc_agg(True)
_sc_agg = _make_sc_agg(False)


def _emb_body(h_ref, w_ref, b_ref, o_ref):
    o_ref[...] = (jnp.dot(h_ref[...], w_ref[...],
                          preferred_element_type=jnp.float32) + b_ref[...])


def _emb(h, wt, b2):
    return pl.pallas_call(
        _emb_body,
        grid=(NB,),
        in_specs=[pl.BlockSpec((ROW_BLK, D), lambda i: (i, 0)),
                  pl.BlockSpec((D, D), lambda i: (0, 0)),
                  pl.BlockSpec((1, D), lambda i: (0, 0))],
        out_specs=pl.BlockSpec((ROW_BLK, D), lambda i: (i, 0)),
        out_shape=jax.ShapeDtypeStruct((N, D), jnp.float32),
    )(h, wt, b2)


def _sage_body(x_ref, agg_ref, deg_ref, wh_ref, wc_ref, b_ref, o_ref):
    x = x_ref[...]
    agg = agg_ref[0] + agg_ref[1]
    deg = deg_ref[0, :, 0:1] + deg_ref[1, :, 0:1]
    cagg = agg / jnp.maximum(deg, 1.0)
    bu = (jnp.dot(x, wh_ref[...], preferred_element_type=jnp.float32)
          + jnp.dot(cagg, wc_ref[...], preferred_element_type=jnp.float32)
          + b_ref[...])
    nrm = jnp.sqrt(jnp.sum(bu * bu, axis=1, keepdims=True))
    bu = bu / jnp.maximum(nrm, 1e-12)
    o_ref[...] = x + jnp.maximum(bu, 0.0)


def _sage(x, aggp, degp, wht, wct, b2):
    return pl.pallas_call(
        _sage_body,
        grid=(NB,),
        in_specs=[pl.BlockSpec((ROW_BLK, D), lambda i: (i, 0)),
                  pl.BlockSpec((NC, ROW_BLK, D), lambda i: (0, i, 0)),
                  pl.BlockSpec((NC, ROW_BLK, 16), lambda i: (0, i, 0)),
                  pl.BlockSpec((D, D), lambda i: (0, 0)),
                  pl.BlockSpec((D, D), lambda i: (0, 0)),
                  pl.BlockSpec((1, D), lambda i: (0, 0))],
        out_specs=pl.BlockSpec((ROW_BLK, D), lambda i: (i, 0)),
        out_shape=jax.ShapeDtypeStruct((N, D), jnp.float32),
    )(x, aggp, degp, wht, wct, b2)


def _head_body(x_ref, gid_ref, pp_ref, pn_ref, wfc_ref, o_ref,
               hg_ref, cnt_ref):
    i = pl.program_id(0)

    @pl.when(i == 0)
    def _():
        hg_ref[...] = jnp.zeros_like(hg_ref)
        cnt_ref[...] = jnp.zeros_like(cnt_ref)

    gid = gid_ref[0]
    lanes = jax.lax.broadcasted_iota(jnp.int32, (B, ROW_BLK), 0)
    onehot_t = (lanes == gid).astype(jnp.float32)
    hg_ref[...] += jnp.dot(onehot_t, x_ref[...],
                           preferred_element_type=jnp.float32)
    cnt_ref[...] += jnp.sum(onehot_t, axis=1, keepdims=True)

    @pl.when(i == NB - 1)
    def _():
        hg = hg_ref[...] / jnp.maximum(cnt_ref[...], 1.0)

        def sims(prots):
            diff = hg[:, None, :] - prots[None, :, :]
            d = jnp.sum(diff * diff, axis=-1)
            return jnp.log((d + 1.0) / (d + 1e-12))

        ss = jnp.concatenate([sims(pp_ref[...]), sims(pn_ref[...])], axis=1)
        y = jnp.dot(ss, wfc_ref[...], preferred_element_type=jnp.float32)
        o_ref[...] = jax.nn.sigmoid(y)


def _head(x, gid3, p_pos, p_neg, wfct):
    p = p_pos.shape[0]
    return pl.pallas_call(
        _head_body,
        grid=(NB,),
        in_specs=[pl.BlockSpec((ROW_BLK, D), lambda i: (i, 0)),
                  pl.BlockSpec((1, 1, ROW_BLK), lambda i: (i, 0, 0)),
                  pl.BlockSpec((p, D), lambda i: (0, 0)),
                  pl.BlockSpec((p, D), lambda i: (0, 0)),
                  pl.BlockSpec((2 * p, 1), lambda i: (0, 0))],
        out_specs=pl.BlockSpec((B, 1), lambda i: (0, 0)),
        out_shape=jax.ShapeDtypeStruct((B, 1), jnp.float32),
        scratch_shapes=[pltpu.VMEM((B, D), jnp.float32),
                        pltpu.VMEM((B, 1), jnp.float32)],
    )(x, gid3, p_pos, p_neg, wfct)


def kernel(h, edge_index, e, graph_ids, W_emb, b_emb, W0, b0, W1, b1,
           p_pos, p_neg, W_fc):
    del e
    srcr = edge_index[0].reshape(NW, NCHUNK, CH)
    dstr = edge_index[1].reshape(NW, NCHUNK, CH)
    z128 = jnp.zeros((ZCH, D), jnp.float32)
    z16 = jnp.zeros((ZCH, 16), jnp.float32)
    ones16 = jnp.ones((CH, 16), jnp.float32)

    x = _emb(h, W_emb.T, b_emb.reshape(1, D))
    agg1, degp = _sc_agg_deg(x, srcr, dstr, z128, z16, ones16)
    x = _sage(x, agg1, degp, W0[:, :D].T, W0[:, D:].T, b0.reshape(1, D))
    (agg2,) = _sc_agg(x, srcr, dstr, z128)
    x = _sage(x, agg2, degp, W1[:, :D].T, W1[:, D:].T, b1.reshape(1, D))

    gid3 = graph_ids.reshape(NB, 1, ROW_BLK)
    y = _head(x, gid3, p_pos, p_neg, W_fc.T)
    return y.reshape(B)


# speedup vs baseline: 6.1239x; 6.1239x over previous
import functools

import jax
import jax.numpy as jnp
from jax import lax
from jax.experimental import pallas as pl
from jax.experimental.pallas import tpu as pltpu
from jax.experimental.pallas import tpu_sc as plsc

N = 10000
E = 320000
D = 128
H = D // 2
B = 64

NC = 2
NS = 16
NW = NC * NS
E_PER = E // NW
CH = 40
NCHUNK = E_PER // CH
ROWS_PER_TILE = 624
TAIL_ROWS = N - NS * ROWS_PER_TILE
ZCH = 104

ROW_BLK = 2000
NB = N // ROW_BLK


def _sc_body(x0_hbm, x1_hbm, srcr, dstr, zh, z16h, onesh,
             agg_o0, agg_o1, deg_o,
             src_v, dst_v, rows0, rows1, zv, acc, s0, s1,
             ones_v, z16v, dacc):
    c = lax.axis_index("c")
    s = lax.axis_index("s")
    wid = s * NC + c
    base_row = s * ROWS_PER_TILE
    tail_base = NS * ROWS_PER_TILE

    pltpu.sync_copy(zh, zv)
    pltpu.sync_copy(z16h, z16v)
    pltpu.sync_copy(onesh, ones_v)
    pltpu.sync_copy(srcr.at[wid], src_v)
    pltpu.sync_copy(dstr.at[wid], dst_v)

    def zero_acc(ref, staging):
        for j in range(ROWS_PER_TILE // ZCH):
            pltpu.sync_copy(staging, ref.at[pl.ds(base_row + j * ZCH, ZCH)])

        @pl.when(s == NS - 1)
        def _():
            pltpu.sync_copy(staging.at[pl.ds(0, TAIL_ROWS)],
                            ref.at[pl.ds(tail_base, TAIL_ROWS)])

    def writeback(ref, out):
        pltpu.sync_copy(ref.at[pl.ds(base_row, ROWS_PER_TILE)],
                        out.at[c, pl.ds(base_row, ROWS_PER_TILE)])

        @pl.when(s == NS - 1)
        def _():
            pltpu.sync_copy(ref.at[pl.ds(tail_base, TAIL_ROWS)],
                            out.at[c, pl.ds(tail_base, TAIL_ROWS)])

    for phase in range(2):
        table = x0_hbm if phase == 0 else x1_hbm
        out = agg_o0 if phase == 0 else agg_o1
        with_deg = phase == 0

        zero_acc(acc, zv)
        if with_deg:
            zero_acc(dacc, z16v)
        plsc.subcore_barrier()

        def fire(chunk, buf, sem):
            pltpu.make_async_copy(table.at[src_v.at[chunk]], buf, sem).start()

        def drain(buf, sem):
            pltpu.make_async_copy(table.at[src_v.at[0]], buf, sem).wait()

        def put(chunk, buf):
            pltpu.sync_copy(buf, acc.at[dst_v.at[chunk]], add=True)
            if with_deg:
                pltpu.sync_copy(ones_v, dacc.at[dst_v.at[chunk]], add=True)

        fire(0, rows0, s0)

        def lbody(i, carry):
            a = 2 * i
            b = a + 1
            fire(b, rows1, s1)
            drain(rows0, s0)
            put(a, rows0)

            @pl.when(b + 1 < NCHUNK)
            def _():
                fire(b + 1, rows0, s0)

            drain(rows1, s1)
            put(b, rows1)
            return carry

        lax.fori_loop(0, NCHUNK // 2, lbody, 0)

        plsc.subcore_barrier()

        writeback(acc, out)
        if with_deg:
            writeback(dacc, deg_o)


@functools.lru_cache(maxsize=None)
def _get_sc_agg():
    mesh = plsc.VectorSubcoreMesh(core_axis_name="c", subcore_axis_name="s")
    out_type = [jax.ShapeDtypeStruct((NC, N, H), jnp.float32),
                jax.ShapeDtypeStruct((NC, N, H), jnp.float32),
                jax.ShapeDtypeStruct((NC, N, 16), jnp.float32)]
    scratch = [
        pltpu.VMEM((NCHUNK, CH), jnp.int32),
        pltpu.VMEM((NCHUNK, CH), jnp.int32),
        pltpu.VMEM((CH, H), jnp.float32),
        pltpu.VMEM((CH, H), jnp.float32),
        pltpu.VMEM((ZCH, H), jnp.float32),
        pltpu.VMEM_SHARED((N, H), jnp.float32),
        pltpu.SemaphoreType.DMA,
        pltpu.SemaphoreType.DMA,
        pltpu.VMEM((CH, 16), jnp.float32),
        pltpu.VMEM((ZCH, 16), jnp.float32),
        pltpu.VMEM_SHARED((N, 16), jnp.float32),
    ]
    return pl.kernel(
        _sc_body, out_type=out_type, mesh=mesh, scratch_types=scratch,
        compiler_params=pltpu.CompilerParams(use_tc_tiling_on_sc=False))


def _sc_agg(x0, x1, srcr, dstr, zh, z16, ones16):
    return _get_sc_agg()(x0, x1, srcr, dstr, zh, z16, ones16)


def _emb_body(h_ref, w_ref, b_ref, o0_ref, o1_ref):
    bu = (jnp.dot(h_ref[...], w_ref[...],
                  preferred_element_type=jnp.float32) + b_ref[...])
    o0_ref[...] = bu[:, :H]
    o1_ref[...] = bu[:, H:]


def _emb(h, wt, b2):
    return pl.pallas_call(
        _emb_body,
        grid=(NB,),
        in_specs=[pl.BlockSpec((ROW_BLK, D), lambda i: (i, 0)),
                  pl.BlockSpec((D, D), lambda i: (0, 0)),
                  pl.BlockSpec((1, D), lambda i: (0, 0))],
        out_specs=[pl.BlockSpec((ROW_BLK, H), lambda i: (i, 0)),
                   pl.BlockSpec((ROW_BLK, H), lambda i: (i, 0))],
        out_shape=[jax.ShapeDtypeStruct((N, H), jnp.float32),
                   jax.ShapeDtypeStruct((N, H), jnp.float32)],
    )(h, wt, b2)


def _sage_body(x0_ref, x1_ref, a0_ref, a1_ref, deg_ref,
               wh_ref, wc_ref, b_ref, o0_ref, o1_ref):
    x0 = x0_ref[...]
    x1 = x1_ref[...]
    deg = deg_ref[0, :, 0:1] + deg_ref[1, :, 0:1]
    inv = 1.0 / jnp.maximum(deg, 1.0)
    c0 = (a0_ref[0] + a0_ref[1]) * inv
    c1 = (a1_ref[0] + a1_ref[1]) * inv
    bu = (jnp.dot(x0, wh_ref[:H], preferred_element_type=jnp.float32)
          + jnp.dot(x1, wh_ref[H:], preferred_element_type=jnp.float32)
          + jnp.dot(c0, wc_ref[:H], preferred_element_type=jnp.float32)
          + jnp.dot(c1, wc_ref[H:], preferred_element_type=jnp.float32)
          + b_ref[...])
    nrm = jnp.sqrt(jnp.sum(bu * bu, axis=1, keepdims=True))
    bu = jnp.maximum(bu / jnp.maximum(nrm, 1e-12), 0.0)
    o0_ref[...] = x0 + bu[:, :H]
    o1_ref[...] = x1 + bu[:, H:]


def _sage(x0, x1, a0, a1, degp, wht, wct, b2):
    return pl.pallas_call(
        _sage_body,
        grid=(NB,),
        in_specs=[pl.BlockSpec((ROW_BLK, H), lambda i: (i, 0)),
                  pl.BlockSpec((ROW_BLK, H), lambda i: (i, 0)),
                  pl.BlockSpec((NC, ROW_BLK, H), lambda i: (0, i, 0)),
                  pl.BlockSpec((NC, ROW_BLK, H), lambda i: (0, i, 0)),
                  pl.BlockSpec((NC, ROW_BLK, 16), lambda i: (0, i, 0)),
                  pl.BlockSpec((D, D), lambda i: (0, 0)),
                  pl.BlockSpec((D, D), lambda i: (0, 0)),
                  pl.BlockSpec((1, D), lambda i: (0, 0))],
        out_specs=[pl.BlockSpec((ROW_BLK, H), lambda i: (i, 0)),
                   pl.BlockSpec((ROW_BLK, H), lambda i: (i, 0))],
        out_shape=[jax.ShapeDtypeStruct((N, H), jnp.float32),
                   jax.ShapeDtypeStruct((N, H), jnp.float32)],
    )(x0, x1, a0, a1, degp, wht, wct, b2)


def _head_body(x0_ref, x1_ref, gid_ref, pp_ref, pn_ref, wfc_ref, o_ref,
               hg_ref, cnt_ref):
    i = pl.program_id(0)

    @pl.when(i == 0)
    def _():
        hg_ref[...] = jnp.zeros_like(hg_ref)
        cnt_ref[...] = jnp.zeros_like(cnt_ref)

    x = jnp.concatenate([x0_ref[...], x1_ref[...]], axis=1)
    gid = gid_ref[0]
    lanes = jax.lax.broadcasted_iota(jnp.int32, (B, ROW_BLK), 0)
    onehot_t = (lanes == gid).astype(jnp.float32)
    hg_ref[...] += jnp.dot(onehot_t, x, preferred_element_type=jnp.float32)
    cnt_ref[...] += jnp.sum(onehot_t, axis=1, keepdims=True)

    @pl.when(i == NB - 1)
    def _():
        hg = hg_ref[...] / jnp.maximum(cnt_ref[...], 1.0)

        def sims(prots):
            diff = hg[:, None, :] - prots[None, :, :]
            d = jnp.sum(diff * diff, axis=-1)
            return jnp.log((d + 1.0) / (d + 1e-12))

        ss = jnp.concatenate([sims(pp_ref[...]), sims(pn_ref[...])], axis=1)
        y = jnp.dot(ss, wfc_ref[...], preferred_element_type=jnp.float32)
        o_ref[...] = jax.nn.sigmoid(y)


def _head(x0, x1, gid3, p_pos, p_neg, wfct):
    p = p_pos.shape[0]
    return pl.pallas_call(
        _head_body,
        grid=(NB,),
        in_specs=[pl.BlockSpec((ROW_BLK, H), lambda i: (i, 0)),
                  pl.BlockSpec((ROW_BLK, H), lambda i: (i, 0)),
                  pl.BlockSpec((1, 1, ROW_BLK), lambda i: (i, 0, 0)),
                  pl.BlockSpec((p, D), lambda i: (0, 0)),
                  pl.BlockSpec((p, D), lambda i: (0, 0)),
                  pl.BlockSpec((2 * p, 1), lambda i: (0, 0))],
        out_specs=pl.BlockSpec((B, 1), lambda i: (0, 0)),
        out_shape=jax.ShapeDtypeStruct((B, 1), jnp.float32),
        scratch_shapes=[pltpu.VMEM((B, D), jnp.float32),
                        pltpu.VMEM((B, 1), jnp.float32)],
    )(x0, x1, gid3, p_pos, p_neg, wfct)


def kernel(h, edge_index, e, graph_ids, W_emb, b_emb, W0, b0, W1, b1,
           p_pos, p_neg, W_fc):
    del e
    srcr = edge_index[0].reshape(NW, NCHUNK, CH)
    dstr = edge_index[1].reshape(NW, NCHUNK, CH)
    zh = jnp.zeros((ZCH, H), jnp.float32)
    z16 = jnp.zeros((ZCH, 16), jnp.float32)
    ones16 = jnp.ones((CH, 16), jnp.float32)

    x0, x1 = _emb(h, W_emb.T, b_emb.reshape(1, D))
    a0, a1, degp = _sc_agg(x0, x1, srcr, dstr, zh, z16, ones16)
    x0, x1 = _sage(x0, x1, a0, a1, degp,
                   W0[:, :D].T, W0[:, D:].T, b0.reshape(1, D))
    a0, a1, _ = _sc_agg(x0, x1, srcr, dstr, zh, z16, ones16)
    x0, x1 = _sage(x0, x1, a0, a1, degp,
                   W1[:, :D].T, W1[:, D:].T, b1.reshape(1, D))

    gid3 = graph_ids.reshape(NB, 1, ROW_BLK)
    y = _head(x0, x1, gid3, p_pos, p_neg, W_fc.T)
    return y.reshape(B)
